# Initial kernel scaffold; baseline (speedup 1.0000x reference)
import jax
import jax.numpy as jnp
from jax.experimental import pallas as pl


def kernel(x_0, node_idx, edge_idx, W01_0, W10_0, b1_0, b0_0, W01_1, W10_1, b1_1, b0_1, W_lin, b_lin):
    raise NotImplementedError("write your pallas kernel here")


# speedup vs baseline: 4.9758x; 4.9758x over previous
import functools

import jax
import jax.numpy as jnp
from jax import lax
from jax.experimental import pallas as pl
from jax.experimental.pallas import tpu as pltpu
from jax.experimental.pallas import tpu_sc as plsc

NN = 10000
NE = 2500
NNZ = 320000
HID = 256
HALF = 128
ALPHA = -1.5
BETA = -0.5
F32 = jnp.float32

RN = 80
RE = 24
CHUNKS = 160
K = 128
PADNNZ = 16 * CHUNKS * K
IDXB = 16
RDN = 10240
RDE = 2560
NW = 32
PERW = NNZ // NW


def _mesh():
    return plsc.VectorSubcoreMesh(core_axis_name="c", subcore_axis_name="s")


def _split16(x):
    return lax.shift_right_logical(x, 7), jnp.bitwise_and(x, 127)


@functools.partial(
    pl.kernel,
    out_type=(
        jax.ShapeDtypeStruct((2, 16, RN * 128), F32),
        jax.ShapeDtypeStruct((2, 16, RE * 128), F32),
    ),
    mesh=_mesh(),
    compiler_params=pltpu.CompilerParams(needs_layout_passes=False),
    scratch_types=[
        pltpu.VMEM((PERW,), jnp.int32),
        pltpu.VMEM((PERW,), jnp.int32),
        pltpu.VMEM((RN * 128,), F32),
        pltpu.VMEM((RE * 128,), F32),
    ],
)
def _deg(nidx, eidx, zeros1, outn, oute, nvm, evm, dn, de):
    c = lax.axis_index("c")
    s = lax.axis_index("s")
    w = s * 2 + c
    pltpu.sync_copy(nidx.at[w], nvm)
    pltpu.sync_copy(eidx.at[w], evm)
    pltpu.sync_copy(zeros1.at[pl.ds(0, RN * 128)], dn)
    pltpu.sync_copy(zeros1.at[pl.ds(0, RE * 128)], de)
    ones = jnp.ones((16,), F32)

    def it(i, carry):
        nid = nvm[pl.ds(i * 16, 16)]
        eid = evm[pl.ds(i * 16, 16)]
        plsc.addupdate_scatter(dn, [nid], ones)
        plsc.addupdate_scatter(de, [eid], ones)
        return carry

    lax.fori_loop(0, PERW // 16, it, 0)
    pltpu.sync_copy(dn, outn.at[c, s])
    pltpu.sync_copy(de, oute.at[c, s])


@functools.partial(
    pl.kernel,
    out_type=(
        jax.ShapeDtypeStruct((2, 16, RN * 128), F32),
        jax.ShapeDtypeStruct((2, 16, RE * 128), F32),
    ),
    mesh=_mesh(),
    compiler_params=pltpu.CompilerParams(needs_layout_passes=False),
    scratch_types=[
        pltpu.VMEM((PERW,), jnp.int32),
        pltpu.VMEM((PERW,), jnp.int32),
        pltpu.VMEM((RN * 128,), F32),
        pltpu.VMEM((RE * 128,), F32),
        pltpu.VMEM((RN * 128,), F32),
        pltpu.VMEM((RE * 128,), F32),
    ],
)
def _dsum(nidx, eidx, ncard, ecard, zeros1, outd0, outd1, nvm, evm, ncv, ecv, d0v, d1v):
    c = lax.axis_index("c")
    s = lax.axis_index("s")
    w = s * 2 + c
    pltpu.sync_copy(nidx.at[w], nvm)
    pltpu.sync_copy(eidx.at[w], evm)
    pltpu.sync_copy(ncard, ncv)
    pltpu.sync_copy(ecard, ecv)
    pltpu.sync_copy(zeros1.at[pl.ds(0, RN * 128)], d0v)
    pltpu.sync_copy(zeros1.at[pl.ds(0, RE * 128)], d1v)

    def it(i, carry):
        nid = nvm[pl.ds(i * 16, 16)]
        eid = evm[pl.ds(i * 16, 16)]
        ec = plsc.load_gather(ecv, [eid])
        plsc.addupdate_scatter(d0v, [nid], ec)
        nc = plsc.load_gather(ncv, [nid])
        plsc.addupdate_scatter(d1v, [eid], nc)
        return carry

    lax.fori_loop(0, PERW // 16, it, 0)
    pltpu.sync_copy(d0v, outd0.at[c, s])
    pltpu.sync_copy(d1v, outd1.at[c, s])


def _make_spmm(rd):
    @functools.partial(
        pl.kernel,
        out_type=jax.ShapeDtypeStruct((2, rd, 128), F32),
        mesh=_mesh(),
        compiler_params=pltpu.CompilerParams(needs_layout_passes=False),
        scratch_types=[
            pltpu.VMEM((IDXB, K), jnp.int32),
            pltpu.VMEM((IDXB, K), jnp.int32),
            pltpu.VMEM((K, 128), F32),
            pltpu.VMEM_SHARED((rd, 128), F32),
            pltpu.SemaphoreType.DMA,
        ],
    )
    def spmm(table, gidx, sidx, zeros, out, gvm, svm, rbuf, accum, sem):
        c = lax.axis_index("c")
        s = lax.axis_index("s")
        rpt = rd // 16
        pltpu.sync_copy(zeros.at[pl.ds(0, rpt)], accum.at[pl.ds(s * rpt, rpt)])
        plsc.subcore_barrier()
        tbl = table.at[c]

        def blk(b, carry):
            pltpu.sync_copy(gidx.at[s, pl.ds(b * IDXB, IDXB)], gvm)
            pltpu.sync_copy(sidx.at[s, pl.ds(b * IDXB, IDXB)], svm)

            def it(j, carry2):
                pltpu.async_copy(tbl.at[gvm.at[j]], rbuf, sem).wait()
                pltpu.sync_copy(rbuf, accum.at[svm.at[j]], add=True)
                return carry2

            lax.fori_loop(0, IDXB, it, 0)
            return carry

        lax.fori_loop(0, CHUNKS // IDXB, blk, 0)
        plsc.subcore_barrier()
        pltpu.sync_copy(accum.at[pl.ds(s * rpt, rpt)], out.at[c, pl.ds(s * rpt, rpt)])

    return spmm


_spmm_edge = _make_spmm(RDE)
_spmm_node = _make_spmm(RDN)


def _cards_body(dnp_ref, dep_ref, nc_ref, ec_ref):
    dn = jnp.sum(dnp_ref[...], axis=0)
    de = jnp.sum(dep_ref[...], axis=0)
    dns = jnp.where(dn > 0, dn, 1.0)
    des = jnp.where(de > 0, de, 1.0)
    nc_ref[...] = jnp.exp(BETA * jnp.log(dns))
    ec_ref[...] = jnp.exp(ALPHA * jnp.log(des))


def _tc_cards(dnp, dep):
    return pl.pallas_call(
        _cards_body,
        out_shape=(
            jax.ShapeDtypeStruct((RN, 128), F32),
            jax.ShapeDtypeStruct((RE, 128), F32),
        ),
    )(dnp, dep)


def _dinv_body(d0p_ref, d1p_ref, d0i_ref, d1i_ref):
    d0 = jnp.sum(d0p_ref[...], axis=0)
    d1 = jnp.sum(d1p_ref[...], axis=0)
    d0i_ref[...] = 1.0 / jnp.maximum(d0, 1e-12)
    d1i_ref[...] = 1.0 / jnp.maximum(d1, 1e-12)


def _tc_dinv(d0p, d1p):
    return pl.pallas_call(
        _dinv_body,
        out_shape=(
            jax.ShapeDtypeStruct((RN, 128), F32),
            jax.ShapeDtypeStruct((RE, 128), F32),
        ),
    )(d0p, d1p)


def _l0_body(x_ref, w_ref, nc_ref, out_ref):
    m = jnp.dot(x_ref[...], w_ref[...], preferred_element_type=F32)
    m = m * nc_ref[...]
    out_ref[0] = m[:, :HALF]
    out_ref[1] = m[:, HALF:]


def _tc_l0(x0, w01, nc_col):
    return pl.pallas_call(
        _l0_body,
        out_shape=jax.ShapeDtypeStruct((2, NN, 128), F32),
    )(x0, w01, nc_col)


def _edge_body(agg_ref, d1i_ref, b1_ref, w_ref, ec_ref, out_ref):
    d1i = d1i_ref[...]
    lo = agg_ref[0, :NE, :]
    hi = agg_ref[1, :NE, :]
    b1 = b1_ref[...]
    x1lo = jnp.maximum(lo * d1i + b1[:, :HALF], 0.0)
    x1hi = jnp.maximum(hi * d1i + b1[:, HALF:], 0.0)
    m = jnp.dot(x1lo, w_ref[:HALF, :], preferred_element_type=F32)
    m = m + jnp.dot(x1hi, w_ref[HALF:, :], preferred_element_type=F32)
    m = m * ec_ref[...]
    out_ref[0] = m[:, :HALF]
    out_ref[1] = m[:, HALF:]


def _tc_edge(agg, d1i_col, b1, w10, ec_col):
    return pl.pallas_call(
        _edge_body,
        out_shape=jax.ShapeDtypeStruct((2, NE, 128), F32),
    )(agg, d1i_col, b1, w10, ec_col)


def _node_body(agg_ref, d0i_ref, b0_ref, w_ref, nc_ref, out_ref):
    d0i = d0i_ref[...]
    lo = agg_ref[0, :NN, :]
    hi = agg_ref[1, :NN, :]
    b0 = b0_ref[...]
    x0lo = jnp.maximum(lo * d0i + b0[:, :HALF], 0.0)
    x0hi = jnp.maximum(hi * d0i + b0[:, HALF:], 0.0)
    m = jnp.dot(x0lo, w_ref[:HALF, :], preferred_element_type=F32)
    m = m + jnp.dot(x0hi, w_ref[HALF:, :], preferred_element_type=F32)
    m = m * nc_ref[...]
    out_ref[0] = m[:, :HALF]
    out_ref[1] = m[:, HALF:]


def _tc_node(agg, d0i_col, b0, w01, nc_col):
    return pl.pallas_call(
        _node_body,
        out_shape=jax.ShapeDtypeStruct((2, NN, 128), F32),
    )(agg, d0i_col, b0, w01, nc_col)


def _final_body(agg_ref, d0i_ref, b0_ref, wl_ref, bl_ref, out_ref):
    d0i = d0i_ref[...]
    lo = agg_ref[0, :NN, :]
    hi = agg_ref[1, :NN, :]
    b0 = b0_ref[...]
    x0lo = jnp.maximum(lo * d0i + b0[:, :HALF], 0.0)
    x0hi = jnp.maximum(hi * d0i + b0[:, HALF:], 0.0)
    mxlo = jnp.max(x0lo, axis=0, keepdims=True)
    mxhi = jnp.max(x0hi, axis=0, keepdims=True)
    r = jnp.dot(mxlo, wl_ref[:HALF, :], preferred_element_type=F32)
    r = r + jnp.dot(mxhi, wl_ref[HALF:, :], preferred_element_type=F32)
    out_ref[...] = r + bl_ref[...]


def _tc_final(agg, d0i_col, b0, wl, bl):
    return pl.pallas_call(
        _final_body,
        out_shape=jax.ShapeDtypeStruct((1, 1), F32),
    )(agg, d0i_col, b0, wl, bl)


def kernel(x_0, node_idx, edge_idx, W01_0, W10_0, b1_0, b0_0,
           W01_1, W10_1, b1_1, b0_1, W_lin, b_lin):
    i32 = jnp.int32
    pad = PADNNZ - NNZ
    nidx32 = node_idx.reshape(NW, PERW)
    eidx32 = edge_idx.reshape(NW, PERW)
    nidx_g = jnp.concatenate([node_idx, jnp.zeros((pad,), i32)]).reshape(16, CHUNKS, K)
    nidx_s = jnp.concatenate([node_idx, jnp.full((pad,), NN, i32)]).reshape(16, CHUNKS, K)
    eidx_g = jnp.concatenate([edge_idx, jnp.zeros((pad,), i32)]).reshape(16, CHUNKS, K)
    eidx_s = jnp.concatenate([edge_idx, jnp.full((pad,), NE, i32)]).reshape(16, CHUNKS, K)
    zeros = jnp.zeros((RDN // 16, 128), F32)
    zeros1 = jnp.zeros((RN * 128,), F32)

    dnp, dep = _deg(nidx32, eidx32, zeros1)
    dnp = dnp.reshape(NW, RN, 128)
    dep = dep.reshape(NW, RE, 128)
    ncard, ecard = _tc_cards(dnp, dep)

    d0p, d1p = _dsum(nidx32, eidx32, ncard.reshape(RN * 128), ecard.reshape(RE * 128), zeros1)
    d0i, d1i = _tc_dinv(d0p.reshape(NW, RN, 128), d1p.reshape(NW, RE, 128))
    d0i_col = d0i.reshape(RN * 128, 1)[:NN]
    d1i_col = d1i.reshape(RE * 128, 1)[:NE]
    nc_col = ncard.reshape(RN * 128, 1)[:NN]
    ec_col = ecard.reshape(RE * 128, 1)[:NE]
    b1_0r = b1_0.reshape(1, HID)
    b0_0r = b0_0.reshape(1, HID)
    b1_1r = b1_1.reshape(1, HID)
    b0_1r = b0_1.reshape(1, HID)

    m0 = _tc_l0(x_0, W01_0, nc_col)
    a1 = _spmm_edge(m0, nidx_g, eidx_s, zeros)
    m1 = _tc_edge(a1, d1i_col, b1_0r, W10_0, ec_col)
    a0 = _spmm_node(m1, eidx_g, nidx_s, zeros)
    m2 = _tc_node(a0, d0i_col, b0_0r, W01_1, nc_col)
    a2 = _spmm_edge(m2, nidx_g, eidx_s, zeros)
    m3 = _tc_edge(a2, d1i_col, b1_1r, W10_1, ec_col)
    a3 = _spmm_node(m3, eidx_g, nidx_s, zeros)
    out = _tc_final(a3, d0i_col, b0_1r, W_lin, b_lin.reshape(1, 1))
    return out.reshape(1)
